# Initial kernel scaffold; baseline (speedup 1.0000x reference)
import jax
import jax.numpy as jnp
from jax.experimental import pallas as pl


def kernel(x, edge_index, W, b):
    raise NotImplementedError("write your pallas kernel here")


# speedup vs baseline: 19.8729x; 19.8729x over previous
import functools

import jax
import jax.numpy as jnp
from jax import lax
from jax.experimental import pallas as pl
from jax.experimental.pallas import tpu as pltpu
from jax.experimental.pallas import tpu_sc as plsc

N_NODES = 10000
F = 128
NC, NS, L = 2, 16, 16
NW = NC * NS
N_PAD = 10240
RPT = N_PAD // NS
CHUNK = 128
DEG_W = 16
ZB = 64
ROW_BLK = 256


def _mesh():
    return plsc.VectorSubcoreMesh(core_axis_name="c", subcore_axis_name="s")


def _deg_body(num_chunks, dst_hbm, degp_hbm, dstv, hist, sumb, res, slab):
    c = lax.axis_index("c")
    s = lax.axis_index("s")
    wid = c * NS + s
    zeros16 = jnp.zeros((L,), jnp.float32)
    ones16 = jnp.ones((L,), jnp.float32)

    def zfill(i, carry):
        hist[pl.ds(i * L, L)] = zeros16
        return carry

    lax.fori_loop(0, N_PAD // L, zfill, 0)
    base_n, rem = divmod(num_chunks, NW)
    nchunks = base_n + jnp.where(wid < rem, 1, 0)

    def chunk(i, carry):
        base = (wid + i * NW) * CHUNK
        pltpu.sync_copy(dst_hbm.at[pl.ds(base, CHUNK)], dstv)
        for j in range(CHUNK // L):
            idx = dstv[pl.ds(j * L, L)]
            plsc.addupdate_scatter(hist, [idx], ones16)
        return carry

    lax.fori_loop(0, nchunks, chunk, 0)
    pltpu.sync_copy(hist, slab.at[s])
    plsc.subcore_barrier()
    pltpu.sync_copy(slab.at[:, pl.ds(s * RPT, RPT)], sumb)

    def comb(k, carry):
        acc = sumb[0, pl.ds(k * L, L)]
        for t in range(1, NS):
            acc = acc + sumb[t, pl.ds(k * L, L)]
        idx = lax.iota(jnp.int32, L) * DEG_W + k * (L * DEG_W)
        plsc.store_scatter(res, [idx], acc)
        return carry

    lax.fori_loop(0, RPT // L, comb, 0)
    pltpu.sync_copy(res, degp_hbm.at[c, pl.ds(s * RPT * DEG_W, RPT * DEG_W)])


def _deg_call(dst, num_chunks):
    k = pl.kernel(
        functools.partial(_deg_body, num_chunks),
        out_type=jax.ShapeDtypeStruct((NC, N_PAD * DEG_W), jnp.float32),
        mesh=_mesh(),
        compiler_params=pltpu.CompilerParams(needs_layout_passes=False),
        scratch_types=[
            pltpu.VMEM((CHUNK,), jnp.int32),
            pltpu.VMEM((N_PAD,), jnp.float32),
            pltpu.VMEM((NS, RPT), jnp.float32),
            pltpu.VMEM((RPT * DEG_W,), jnp.float32),
            pltpu.VMEM_SHARED((NS, N_PAD), jnp.float32),
        ],
    )
    return k(dst).reshape(NC, N_PAD, DEG_W)


def _scat_body(num_chunks, hs_hbm, src_hbm, dst_hbm, part_hbm, srcv, dstv,
               rows, stage, acc, sem):
    c = lax.axis_index("c")
    s = lax.axis_index("s")
    wid = c * NS + s
    zeros16 = jnp.zeros((L,), jnp.float32)

    def zero(i, carry):
        for j in range(F // L):
            stage[i, pl.ds(j * L, L)] = zeros16
        return carry

    lax.fori_loop(0, ZB, zero, 0)
    row0 = s * RPT

    def zcopy(k, carry):
        pltpu.sync_copy(stage, acc.at[pl.ds(row0 + k * ZB, ZB)])
        return carry

    lax.fori_loop(0, RPT // ZB, zcopy, 0)
    plsc.subcore_barrier()

    base_n, rem = divmod(num_chunks, NW)
    nchunks = base_n + jnp.where(wid < rem, 1, 0)

    def body(i, carry):
        base = (wid + i * NW) * CHUNK
        pltpu.sync_copy(src_hbm.at[pl.ds(base, CHUNK)], srcv)
        pltpu.sync_copy(dst_hbm.at[pl.ds(base, CHUNK)], dstv)
        pltpu.async_copy(hs_hbm.at[srcv], rows, sem).wait()
        pltpu.sync_copy(rows, acc.at[dstv], add=True)
        return carry

    lax.fori_loop(0, nchunks, body, 0)
    plsc.subcore_barrier()

    def wcopy(k, carry):
        pltpu.sync_copy(acc.at[pl.ds(row0 + k * ZB, ZB)], stage)
        pltpu.sync_copy(stage, part_hbm.at[c, pl.ds(row0 + k * ZB, ZB)])
        return carry

    lax.fori_loop(0, RPT // ZB, wcopy, 0)


def _scat_call(hs, src, dst, num_chunks):
    k = pl.kernel(
        functools.partial(_scat_body, num_chunks),
        out_type=jax.ShapeDtypeStruct((NC, N_PAD, F), jnp.float32),
        mesh=_mesh(),
        scratch_types=[
            pltpu.VMEM((CHUNK,), jnp.int32),
            pltpu.VMEM((CHUNK,), jnp.int32),
            pltpu.VMEM((CHUNK, F), jnp.float32),
            pltpu.VMEM((ZB, F), jnp.float32),
            pltpu.VMEM_SHARED((N_PAD, F), jnp.float32),
            pltpu.SemaphoreType.DMA,
        ],
    )
    return k(hs, src, dst)


def _mm_body(x_ref, w_ref, degp_ref, hs_ref):
    deg = degp_ref[0, :, 0:1] + degp_ref[1, :, 0:1] + 1.0
    dis = lax.rsqrt(deg)
    h = jnp.dot(x_ref[...], w_ref[...], preferred_element_type=jnp.float32)
    hs_ref[...] = h * dis


def _mm_call(x_pad, W, degp):
    grid = (N_PAD // ROW_BLK,)
    return pl.pallas_call(
        _mm_body,
        grid=grid,
        in_specs=[
            pl.BlockSpec((ROW_BLK, F), lambda i: (i, 0)),
            pl.BlockSpec((F, F), lambda i: (0, 0)),
            pl.BlockSpec((NC, ROW_BLK, DEG_W), lambda i: (0, i, 0)),
        ],
        out_specs=pl.BlockSpec((ROW_BLK, F), lambda i: (i, 0)),
        out_shape=jax.ShapeDtypeStruct((N_PAD, F), jnp.float32),
    )(x_pad, W, degp)


def _fin_body(part_ref, hs_ref, degp_ref, b_ref, out_ref):
    deg = degp_ref[0, :, 0:1] + degp_ref[1, :, 0:1] + 1.0
    dis = lax.rsqrt(deg)
    out_ref[...] = dis * (part_ref[0] + part_ref[1] + hs_ref[...]) + b_ref[...]


def _fin_call(part, hs, degp, b2d):
    grid = (N_PAD // ROW_BLK,)
    return pl.pallas_call(
        _fin_body,
        grid=grid,
        in_specs=[
            pl.BlockSpec((NC, ROW_BLK, F), lambda i: (0, i, 0)),
            pl.BlockSpec((ROW_BLK, F), lambda i: (i, 0)),
            pl.BlockSpec((NC, ROW_BLK, DEG_W), lambda i: (0, i, 0)),
            pl.BlockSpec((1, F), lambda i: (0, 0)),
        ],
        out_specs=pl.BlockSpec((ROW_BLK, F), lambda i: (i, 0)),
        out_shape=jax.ShapeDtypeStruct((N_PAD, F), jnp.float32),
    )(part, hs, degp, b2d)


def kernel(x, edge_index, W, b):
    n, f_in = x.shape
    e = edge_index.shape[1]
    assert n == N_NODES and f_in == F and e % CHUNK == 0
    num_chunks = e // CHUNK

    src = edge_index[0]
    dst = edge_index[1]
    x_pad = jnp.pad(x, ((0, N_PAD - N_NODES), (0, 0)))

    degp = _deg_call(dst, num_chunks)
    hs = _mm_call(x_pad, W, degp)
    part = _scat_call(hs, src, dst, num_chunks)
    out = _fin_call(part, hs, degp, b.reshape(1, F))
    return out[:N_NODES]


# speedup vs baseline: 38.9197x; 1.9584x over previous
import functools

import jax
import jax.numpy as jnp
from jax import lax
from jax.experimental import pallas as pl
from jax.experimental.pallas import tpu as pltpu
from jax.experimental.pallas import tpu_sc as plsc

N_NODES = 10000
F = 128
NC, NS, L = 2, 16, 16
NW = NC * NS
N_PAD = 10240
RPT = N_PAD // NS
CHUNK = 128
DEG_W = 16
ZB = 32
IBATCH = 20
ROW_BLK = 2000


def _mesh():
    return plsc.VectorSubcoreMesh(core_axis_name="c", subcore_axis_name="s")


def _deg_body(nch_w, dst_hbm_eidx, degp_hbm, ibuf, hist, sumb, res, slab):
    c = lax.axis_index("c")
    s = lax.axis_index("s")
    wid = c * NS + s
    w0 = wid * nch_w
    zeros16 = jnp.zeros((L,), jnp.float32)
    ones16 = jnp.ones((L,), jnp.float32)

    def zfill(i, carry):
        hist[pl.ds(i * L, L)] = zeros16
        return carry

    lax.fori_loop(0, N_PAD // L, zfill, 0)

    nb = nch_w // IBATCH
    for b in range(nb):
        pltpu.sync_copy(dst_hbm_eidx.at[pl.ds(w0 + b * IBATCH, IBATCH)], ibuf)
        for j in range(IBATCH):
            for k in range(CHUNK // L):
                idx = ibuf[j, 1, pl.ds(k * L, L)]
                plsc.addupdate_scatter(hist, [idx], ones16)

    pltpu.sync_copy(hist, slab.at[s])
    plsc.subcore_barrier()
    pltpu.sync_copy(slab.at[:, pl.ds(s * RPT, RPT)], sumb)

    def comb(k, carry):
        acc = sumb[0, pl.ds(k * L, L)]
        for t in range(1, NS):
            acc = acc + sumb[t, pl.ds(k * L, L)]
        idx = lax.iota(jnp.int32, L) * DEG_W + k * (L * DEG_W)
        plsc.store_scatter(res, [idx], acc)
        return carry

    lax.fori_loop(0, RPT // L, comb, 0)
    pltpu.sync_copy(res, degp_hbm.at[c, pl.ds(s * RPT * DEG_W, RPT * DEG_W)])


def _deg_call(eidx, nch_w):
    k = pl.kernel(
        functools.partial(_deg_body, nch_w),
        out_type=jax.ShapeDtypeStruct((NC, N_PAD * DEG_W), jnp.float32),
        mesh=_mesh(),
        compiler_params=pltpu.CompilerParams(needs_layout_passes=False),
        scratch_types=[
            pltpu.VMEM((IBATCH, 2, CHUNK), jnp.int32),
            pltpu.VMEM((N_PAD,), jnp.float32),
            pltpu.VMEM((NS, RPT), jnp.float32),
            pltpu.VMEM((RPT * DEG_W,), jnp.float32),
            pltpu.VMEM_SHARED((NS, N_PAD), jnp.float32),
        ],
    )
    return k(eidx).reshape(NC, N_PAD, DEG_W)


def _scat_body(nch_w, hs_hbm, eidx_hbm, part_hbm, ibuf, rows0, rows1, stage,
               acc, gsem0, gsem1, isem):
    c = lax.axis_index("c")
    s = lax.axis_index("s")
    wid = c * NS + s
    w0 = wid * nch_w
    zeros16 = jnp.zeros((L,), jnp.float32)

    def zero(i, carry):
        for j in range(F // L):
            stage[i, pl.ds(j * L, L)] = zeros16
        return carry

    lax.fori_loop(0, ZB, zero, 0)
    row0 = s * RPT

    def zcopy(k, carry):
        pltpu.sync_copy(stage, acc.at[pl.ds(row0 + k * ZB, ZB)])
        return carry

    lax.fori_loop(0, RPT // ZB, zcopy, 0)
    plsc.subcore_barrier()

    rows = [rows0, rows1]
    gsem = [gsem0, gsem1]
    nb = nch_w // IBATCH

    pltpu.sync_copy(eidx_hbm.at[pl.ds(w0, IBATCH)], ibuf.at[0])
    gathers = [None, None]
    gathers[0] = pltpu.async_copy(hs_hbm.at[ibuf.at[0, 0, 0]], rows[0], gsem[0])
    iload = None
    total = nb * IBATCH
    for b in range(nb):
        q = b % 2
        if b + 1 < nb:
            iload = pltpu.async_copy(
                eidx_hbm.at[pl.ds(w0 + (b + 1) * IBATCH, IBATCH)],
                ibuf.at[1 - q], isem)
        for j in range(IBATCH):
            g = b * IBATCH + j
            p = g % 2
            if g + 1 < total:
                nb_idx, nj = divmod(g + 1, IBATCH)
                if nj == 0:
                    iload.wait()
                gathers[p].wait()
                gathers[1 - p] = pltpu.async_copy(
                    hs_hbm.at[ibuf.at[nb_idx % 2, nj, 0]], rows[1 - p],
                    gsem[1 - p])
            else:
                gathers[p].wait()
            pltpu.sync_copy(rows[p], acc.at[ibuf.at[q, j, 1]], add=True)

    plsc.subcore_barrier()

    def wcopy(k, carry):
        pltpu.sync_copy(acc.at[pl.ds(row0 + k * ZB, ZB)], stage)
        pltpu.sync_copy(stage, part_hbm.at[c, pl.ds(row0 + k * ZB, ZB)])
        return carry

    lax.fori_loop(0, RPT // ZB, wcopy, 0)


def _scat_call(hs, eidx, nch_w):
    k = pl.kernel(
        functools.partial(_scat_body, nch_w),
        out_type=jax.ShapeDtypeStruct((NC, N_PAD, F), jnp.float32),
        mesh=_mesh(),
        scratch_types=[
            pltpu.VMEM((2, IBATCH, 2, CHUNK), jnp.int32),
            pltpu.VMEM((CHUNK, F), jnp.float32),
            pltpu.VMEM((CHUNK, F), jnp.float32),
            pltpu.VMEM((ZB, F), jnp.float32),
            pltpu.VMEM_SHARED((N_PAD, F), jnp.float32),
            pltpu.SemaphoreType.DMA,
            pltpu.SemaphoreType.DMA,
            pltpu.SemaphoreType.DMA,
        ],
    )
    return k(hs, eidx)


def _mm_body(x_ref, w_ref, degp_ref, hs_ref):
    deg = degp_ref[0, :, 0:1] + degp_ref[1, :, 0:1] + 1.0
    dis = lax.rsqrt(deg)
    h = jnp.dot(x_ref[...], w_ref[...], preferred_element_type=jnp.float32)
    hs_ref[...] = h * dis


def _mm_call(x, W, degp):
    grid = (N_NODES // ROW_BLK,)
    return pl.pallas_call(
        _mm_body,
        grid=grid,
        in_specs=[
            pl.BlockSpec((ROW_BLK, F), lambda i: (i, 0)),
            pl.BlockSpec((F, F), lambda i: (0, 0)),
            pl.BlockSpec((NC, ROW_BLK, DEG_W), lambda i: (0, i, 0)),
        ],
        out_specs=pl.BlockSpec((ROW_BLK, F), lambda i: (i, 0)),
        out_shape=jax.ShapeDtypeStruct((N_PAD, F), jnp.float32),
    )(x, W, degp)


def _fin_body(part_ref, hs_ref, degp_ref, b_ref, out_ref):
    deg = degp_ref[0, :, 0:1] + degp_ref[1, :, 0:1] + 1.0
    dis = lax.rsqrt(deg)
    out_ref[...] = dis * (part_ref[0] + part_ref[1] + hs_ref[...]) + b_ref[...]


def _fin_call(part, hs, degp, b2d):
    grid = (N_NODES // ROW_BLK,)
    return pl.pallas_call(
        _fin_body,
        grid=grid,
        in_specs=[
            pl.BlockSpec((NC, ROW_BLK, F), lambda i: (0, i, 0)),
            pl.BlockSpec((ROW_BLK, F), lambda i: (i, 0)),
            pl.BlockSpec((NC, ROW_BLK, DEG_W), lambda i: (0, i, 0)),
            pl.BlockSpec((1, F), lambda i: (0, 0)),
        ],
        out_specs=pl.BlockSpec((ROW_BLK, F), lambda i: (i, 0)),
        out_shape=jax.ShapeDtypeStruct((N_NODES, F), jnp.float32),
    )(part, hs, degp, b2d)


def kernel(x, edge_index, W, b):
    n, f_in = x.shape
    e = edge_index.shape[1]
    assert n == N_NODES and f_in == F

    nch = -(-e // CHUNK)
    nch_w = -(-nch // NW)
    nch_w = -(-nch_w // IBATCH) * IBATCH
    e_pad = nch_w * NW * CHUNK
    fill = N_NODES + (jnp.arange(e_pad - e, dtype=jnp.int32)
                      % (N_PAD - N_NODES))
    ep = jnp.concatenate(
        [edge_index.astype(jnp.int32), jnp.stack([fill, fill])], axis=1)
    eidx = ep.reshape(2, nch_w * NW, CHUNK).transpose(1, 0, 2)

    degp = _deg_call(eidx, nch_w)
    hs = _mm_call(x, W, degp)
    part = _scat_call(hs, eidx, nch_w)
    out = _fin_call(part, hs, degp, b.reshape(1, F))
    return out
